# Initial kernel scaffold; baseline (speedup 1.0000x reference)
import jax
import jax.numpy as jnp
from jax.experimental import pallas as pl


def kernel(user_id, item_id, user_table, item_table, W1, b1, W2, b2, W3, b3):
    raise NotImplementedError("write your pallas kernel here")


# speedup vs baseline: 2.1695x; 2.1695x over previous
import functools

import jax
import jax.numpy as jnp
from jax import lax
from jax.experimental import pallas as pl
from jax.experimental.pallas import tpu as pltpu
from jax.experimental.pallas import tpu_sc as plsc

B = 16384
D = 64
NC, NS = 2, 16
NW = NC * NS
BPW = B // NW
CHUNK = 128
NCHUNK = BPW // CHUNK

@functools.cache
def _make_sc_gather():
    mesh = plsc.VectorSubcoreMesh(core_axis_name="c", subcore_axis_name="s")

    @functools.partial(
        pl.kernel,
        out_type=(
            jax.ShapeDtypeStruct((B, D), jnp.float32),
            jax.ShapeDtypeStruct((B, D), jnp.float32),
        ),
        mesh=mesh,
        scratch_types=[
            pltpu.VMEM((NCHUNK, CHUNK), jnp.int32),
            pltpu.VMEM((NCHUNK, CHUNK), jnp.int32),
            pltpu.VMEM((NCHUNK, CHUNK, D), jnp.float32),
            pltpu.VMEM((NCHUNK, CHUNK, D), jnp.float32),
            pltpu.SemaphoreType.DMA,
            pltpu.SemaphoreType.DMA,
        ],
        compiler_params=pltpu.CompilerParams(use_tc_tiling_on_sc=False),
    )
    def sc_gather(user_tbl, item_tbl, uid, iid, out_u, out_i,
                  uidx, iidx, urows, irows, usem, isem):
        wid = lax.axis_index("s") * NC + lax.axis_index("c")
        base = wid * BPW
        pltpu.sync_copy(uid.at[wid], uidx)
        pltpu.sync_copy(iid.at[wid], iidx)
        ucopies = []
        icopies = []
        for j in range(NCHUNK):
            ucopies.append(pltpu.async_copy(user_tbl.at[uidx.at[j]], urows.at[j], usem))
            icopies.append(pltpu.async_copy(item_tbl.at[iidx.at[j]], irows.at[j], isem))
        for j in range(NCHUNK):
            ucopies[j].wait()
            icopies[j].wait()
        for j in range(NCHUNK):
            pltpu.sync_copy(urows.at[j], out_u.at[pl.ds(base + j * CHUNK, CHUNK)])
            pltpu.sync_copy(irows.at[j], out_i.at[pl.ds(base + j * CHUNK, CHUNK)])

    return sc_gather


BS = 2048


def _mlp_body(ug, ig, w1, b1, w2, b2, w3t, b3, out):
    h = jnp.dot(ug[:], w1[0:D, :], preferred_element_type=jnp.float32)
    h = h + jnp.dot(ig[:], w1[D:2 * D, :], preferred_element_type=jnp.float32)
    h = jnp.maximum(h + b1[:], 0.0)
    h = jnp.maximum(jnp.dot(h, w2[:], preferred_element_type=jnp.float32) + b2[:], 0.0)
    out[:] = jnp.sum(h * w3t[:], axis=1, keepdims=True) + b3[:]


_mlp_call = pl.pallas_call(
    _mlp_body,
    grid=(B // BS,),
    in_specs=[
        pl.BlockSpec((BS, D), lambda i: (i, 0)),
        pl.BlockSpec((BS, D), lambda i: (i, 0)),
        pl.BlockSpec((2 * D, D), lambda i: (0, 0)),
        pl.BlockSpec((1, D), lambda i: (0, 0)),
        pl.BlockSpec((D, 32), lambda i: (0, 0)),
        pl.BlockSpec((1, 32), lambda i: (0, 0)),
        pl.BlockSpec((1, 32), lambda i: (0, 0)),
        pl.BlockSpec((1, 1), lambda i: (0, 0)),
    ],
    out_specs=pl.BlockSpec((BS, 1), lambda i: (i, 0)),
    out_shape=jax.ShapeDtypeStruct((B, 1), jnp.float32),
)


def kernel(user_id, item_id, user_table, item_table, W1, b1, W2, b2, W3, b3):
    uid = user_id.astype(jnp.int32).reshape(NW, NCHUNK, CHUNK)
    iid = item_id.astype(jnp.int32).reshape(NW, NCHUNK, CHUNK)
    ug, ig = _make_sc_gather()(user_table, item_table, uid, iid)
    return _mlp_call(ug, ig, W1, b1.reshape(1, D), W2, b2.reshape(1, 32),
                     W3.reshape(1, 32), b3.reshape(1, 1))


# speedup vs baseline: 2.5346x; 1.1683x over previous
import functools

import jax
import jax.numpy as jnp
from jax import lax
from jax.experimental import pallas as pl
from jax.experimental.pallas import tpu as pltpu
from jax.experimental.pallas import tpu_sc as plsc

B = 16384
D = 64
NC, NS = 2, 16
NW = NC * NS
BPW = B // NW
CHUNK = 128
NCHUNK = BPW // CHUNK


@functools.cache
def _make_sc_gather():
    mesh = plsc.VectorSubcoreMesh(core_axis_name="c", subcore_axis_name="s")

    @functools.partial(
        pl.kernel,
        out_type=(
            jax.ShapeDtypeStruct((B, D), jnp.float32),
            jax.ShapeDtypeStruct((B, D), jnp.float32),
        ),
        mesh=mesh,
        scratch_types=[
            pltpu.VMEM((NCHUNK, CHUNK), jnp.int32),
            pltpu.VMEM((NCHUNK, CHUNK), jnp.int32),
            pltpu.VMEM((NCHUNK, CHUNK, D), jnp.float32),
            pltpu.VMEM((NCHUNK, CHUNK, D), jnp.float32),
            pltpu.SemaphoreType.DMA,
            pltpu.SemaphoreType.DMA,
        ],
        compiler_params=pltpu.CompilerParams(use_tc_tiling_on_sc=False),
    )
    def sc_gather(user_tbl, item_tbl, uid, iid, out_u, out_i,
                  uidx, iidx, urows, irows, usem, isem):
        wid = lax.axis_index("s") * NC + lax.axis_index("c")
        base = wid * BPW
        for j in range(NCHUNK):
            pltpu.sync_copy(uid.at[pl.ds(base + j * CHUNK, CHUNK)], uidx.at[j])
            pltpu.sync_copy(iid.at[pl.ds(base + j * CHUNK, CHUNK)], iidx.at[j])
        ucopies = []
        icopies = []
        for j in range(NCHUNK):
            ucopies.append(pltpu.async_copy(user_tbl.at[uidx.at[j]], urows.at[j], usem))
            icopies.append(pltpu.async_copy(item_tbl.at[iidx.at[j]], irows.at[j], isem))
        for j in range(NCHUNK):
            ucopies[j].wait()
            icopies[j].wait()
        for j in range(NCHUNK):
            pltpu.sync_copy(urows.at[j], out_u.at[pl.ds(base + j * CHUNK, CHUNK)])
            pltpu.sync_copy(irows.at[j], out_i.at[pl.ds(base + j * CHUNK, CHUNK)])

    return sc_gather


BS2 = 1024


def _mlp_body(up, ip, w1, b1, w2, b2, w3t, b3, out):
    w1u = w1[0:D, :]
    w1i = w1[D:2 * D, :]
    he = jnp.dot(up[:, 0:D], w1u, preferred_element_type=jnp.float32)
    he = he + jnp.dot(ip[:, 0:D], w1i, preferred_element_type=jnp.float32)
    he = jnp.maximum(he + b1[:], 0.0)
    ho = jnp.dot(up[:, D:2 * D], w1u, preferred_element_type=jnp.float32)
    ho = ho + jnp.dot(ip[:, D:2 * D], w1i, preferred_element_type=jnp.float32)
    ho = jnp.maximum(ho + b1[:], 0.0)
    he = jnp.maximum(jnp.dot(he, w2[:], preferred_element_type=jnp.float32) + b2[:], 0.0)
    ho = jnp.maximum(jnp.dot(ho, w2[:], preferred_element_type=jnp.float32) + b2[:], 0.0)
    oe = jnp.sum(he * w3t[:], axis=1, keepdims=True) + b3[:]
    oo = jnp.sum(ho * w3t[:], axis=1, keepdims=True) + b3[:]
    out[:] = jnp.concatenate([oe, oo], axis=1)


_mlp_call = pl.pallas_call(
    _mlp_body,
    grid=(B // 2 // BS2,),
    in_specs=[
        pl.BlockSpec((BS2, 2 * D), lambda i: (i, 0)),
        pl.BlockSpec((BS2, 2 * D), lambda i: (i, 0)),
        pl.BlockSpec((2 * D, D), lambda i: (0, 0)),
        pl.BlockSpec((1, D), lambda i: (0, 0)),
        pl.BlockSpec((D, 32), lambda i: (0, 0)),
        pl.BlockSpec((1, 32), lambda i: (0, 0)),
        pl.BlockSpec((1, 32), lambda i: (0, 0)),
        pl.BlockSpec((1, 1), lambda i: (0, 0)),
    ],
    out_specs=pl.BlockSpec((BS2, 2), lambda i: (i, 0)),
    out_shape=jax.ShapeDtypeStruct((B // 2, 2), jnp.float32),
)


def kernel(user_id, item_id, user_table, item_table, W1, b1, W2, b2, W3, b3):
    uid = user_id.astype(jnp.int32)
    iid = item_id.astype(jnp.int32)
    ug, ig = _make_sc_gather()(user_table, item_table, uid, iid)
    up = ug.reshape(B // 2, 2 * D)
    ip = ig.reshape(B // 2, 2 * D)
    out2 = _mlp_call(up, ip, W1, b1.reshape(1, D), W2, b2.reshape(1, 32),
                     W3.reshape(1, 32), b3.reshape(1, 1))
    return out2.reshape(B, 1)


# speedup vs baseline: 3.1673x; 1.2496x over previous
import functools

import jax
import jax.numpy as jnp
from jax import lax
from jax.experimental import pallas as pl
from jax.experimental.pallas import tpu as pltpu
from jax.experimental.pallas import tpu_sc as plsc

B = 16384
D = 64
NC, NS = 2, 16
NW = NC * NS
BPW = B // NW
CHUNK = 128
NCHUNK = BPW // CHUNK


@functools.cache
def _make_sc_gather():
    mesh = plsc.VectorSubcoreMesh(core_axis_name="c", subcore_axis_name="s")

    @functools.partial(
        pl.kernel,
        out_type=jax.ShapeDtypeStruct((B, 2 * D), jnp.float32),
        mesh=mesh,
        scratch_types=[
            pltpu.VMEM((NCHUNK, CHUNK), jnp.int32),
            pltpu.VMEM((NCHUNK, CHUNK), jnp.int32),
            pltpu.VMEM((NCHUNK, CHUNK, D), jnp.float32),
            pltpu.VMEM((NCHUNK, CHUNK, D), jnp.float32),
            pltpu.SemaphoreType.DMA,
            pltpu.SemaphoreType.DMA,
        ],
        compiler_params=pltpu.CompilerParams(use_tc_tiling_on_sc=False),
    )
    def sc_gather(user_tbl, item_tbl, uid, iid, out,
                  uidx, iidx, urows, irows, usem, isem):
        wid = lax.axis_index("s") * NC + lax.axis_index("c")
        base = wid * BPW
        for j in range(NCHUNK):
            pltpu.sync_copy(uid.at[pl.ds(base + j * CHUNK, CHUNK)], uidx.at[j])
            pltpu.sync_copy(iid.at[pl.ds(base + j * CHUNK, CHUNK)], iidx.at[j])
        ucopies = []
        icopies = []
        for j in range(NCHUNK):
            ucopies.append(pltpu.async_copy(user_tbl.at[uidx.at[j]], urows.at[j], usem))
            icopies.append(pltpu.async_copy(item_tbl.at[iidx.at[j]], irows.at[j], isem))
        for j in range(NCHUNK):
            ucopies[j].wait()
            icopies[j].wait()
        for j in range(NCHUNK):
            rows = pl.ds(base + j * CHUNK, CHUNK)
            pltpu.sync_copy(urows.at[j], out.at[rows, pl.ds(0, D)])
            pltpu.sync_copy(irows.at[j], out.at[rows, pl.ds(D, D)])

    return sc_gather


BS = 2048
OROWS = BS // 128


def _mlp_body(x, w1, b1, w2, b2, w3t, b3, out):
    h = jnp.dot(x[:], w1[:], preferred_element_type=jnp.float32)
    h = jnp.maximum(h + b1[:], 0.0)
    h = jnp.maximum(jnp.dot(h, w2[:], preferred_element_type=jnp.float32) + b2[:], 0.0)
    ot = lax.dot_general(w3t[:], h, (((1,), (1,)), ((), ())),
                         preferred_element_type=jnp.float32) + b3[:]
    for r in range(OROWS):
        out[r:r + 1, :] = ot[:, r * 128:(r + 1) * 128]


_mlp_call = pl.pallas_call(
    _mlp_body,
    grid=(B // BS,),
    in_specs=[
        pl.BlockSpec((BS, 2 * D), lambda i: (i, 0)),
        pl.BlockSpec((2 * D, D), lambda i: (0, 0)),
        pl.BlockSpec((1, D), lambda i: (0, 0)),
        pl.BlockSpec((D, 32), lambda i: (0, 0)),
        pl.BlockSpec((1, 32), lambda i: (0, 0)),
        pl.BlockSpec((1, 32), lambda i: (0, 0)),
        pl.BlockSpec((1, 1), lambda i: (0, 0)),
    ],
    out_specs=pl.BlockSpec((OROWS, 128), lambda i: (i, 0)),
    out_shape=jax.ShapeDtypeStruct((B // 128, 128), jnp.float32),
)


def kernel(user_id, item_id, user_table, item_table, W1, b1, W2, b2, W3, b3):
    uid = user_id.astype(jnp.int32)
    iid = item_id.astype(jnp.int32)
    xcat = _make_sc_gather()(user_table, item_table, uid, iid)
    outp = _mlp_call(xcat, W1, b1.reshape(1, D), W2, b2.reshape(1, 32),
                     W3.reshape(1, 32), b3.reshape(1, 1))
    return outp.reshape(B, 1)


# speedup vs baseline: 3.3896x; 1.0702x over previous
import functools

import jax
import jax.numpy as jnp
from jax import lax
from jax.experimental import pallas as pl
from jax.experimental.pallas import tpu as pltpu
from jax.experimental.pallas import tpu_sc as plsc

B = 16384
D = 64
NC, NS = 2, 16
NW = NC * NS
BPW = B // NW
CHUNK = 128
NCHUNK = BPW // CHUNK


@functools.cache
def _make_sc_gather():
    mesh = plsc.VectorSubcoreMesh(core_axis_name="c", subcore_axis_name="s")

    @functools.partial(
        pl.kernel,
        out_type=jax.ShapeDtypeStruct((B, 2 * D), jnp.float32),
        mesh=mesh,
        scratch_types=[
            pltpu.VMEM((NCHUNK, CHUNK), jnp.int32),
            pltpu.VMEM((NCHUNK, CHUNK), jnp.int32),
            pltpu.VMEM((NCHUNK, CHUNK, D), jnp.float32),
            pltpu.VMEM((NCHUNK, CHUNK, D), jnp.float32),
            pltpu.SemaphoreType.DMA,
            pltpu.SemaphoreType.DMA,
            pltpu.SemaphoreType.DMA,
            pltpu.SemaphoreType.DMA,
        ],
        compiler_params=pltpu.CompilerParams(use_tc_tiling_on_sc=False),
    )
    def sc_gather(user_tbl, item_tbl, uid, iid, out,
                  uidx, iidx, urows, irows, dsem, usem, isem, wsem):
        wid = lax.axis_index("s") * NC + lax.axis_index("c")
        base = wid * BPW
        idxc = []
        for j in range(NCHUNK):
            idxc.append(pltpu.async_copy(
                uid.at[pl.ds(base + j * CHUNK, CHUNK)], uidx.at[j], dsem))
            idxc.append(pltpu.async_copy(
                iid.at[pl.ds(base + j * CHUNK, CHUNK)], iidx.at[j], dsem))
        for c in idxc:
            c.wait()
        ucopies = []
        icopies = []
        for j in range(NCHUNK):
            ucopies.append(pltpu.async_copy(user_tbl.at[uidx.at[j]], urows.at[j], usem))
            icopies.append(pltpu.async_copy(item_tbl.at[iidx.at[j]], irows.at[j], isem))
        writes = []
        for j in range(NCHUNK):
            rows = pl.ds(base + j * CHUNK, CHUNK)
            ucopies[j].wait()
            writes.append(pltpu.async_copy(urows.at[j], out.at[rows, pl.ds(0, D)], wsem))
            icopies[j].wait()
            writes.append(pltpu.async_copy(irows.at[j], out.at[rows, pl.ds(D, D)], wsem))
        for w in writes:
            w.wait()

    return sc_gather


BS = 2048
OROWS = BS // 128


def _mlp_body(x, w1, b1, w2, b2, w3t, b3, out):
    h = jnp.dot(x[:], w1[:], preferred_element_type=jnp.float32)
    h = jnp.maximum(h + b1[:], 0.0)
    h = jnp.maximum(jnp.dot(h, w2[:], preferred_element_type=jnp.float32) + b2[:], 0.0)
    ot = lax.dot_general(w3t[:], h, (((1,), (1,)), ((), ())),
                         preferred_element_type=jnp.float32) + b3[:]
    for r in range(OROWS):
        out[r:r + 1, :] = ot[:, r * 128:(r + 1) * 128]


_mlp_call = pl.pallas_call(
    _mlp_body,
    grid=(B // BS,),
    in_specs=[
        pl.BlockSpec((BS, 2 * D), lambda i: (i, 0)),
        pl.BlockSpec((2 * D, D), lambda i: (0, 0)),
        pl.BlockSpec((1, D), lambda i: (0, 0)),
        pl.BlockSpec((D, 32), lambda i: (0, 0)),
        pl.BlockSpec((1, 32), lambda i: (0, 0)),
        pl.BlockSpec((1, 32), lambda i: (0, 0)),
        pl.BlockSpec((1, 1), lambda i: (0, 0)),
    ],
    out_specs=pl.BlockSpec((OROWS, 128), lambda i: (i, 0)),
    out_shape=jax.ShapeDtypeStruct((B // 128, 128), jnp.float32),
    compiler_params=pltpu.CompilerParams(dimension_semantics=("parallel",)),
)


def kernel(user_id, item_id, user_table, item_table, W1, b1, W2, b2, W3, b3):
    uid = user_id.astype(jnp.int32)
    iid = item_id.astype(jnp.int32)
    xcat = _make_sc_gather()(user_table, item_table, uid, iid)
    outp = _mlp_call(xcat, W1, b1.reshape(1, D), W2, b2.reshape(1, 32),
                     W3.reshape(1, 32), b3.reshape(1, 1))
    return outp.reshape(B, 1)
